# baseline (speedup 1.0000x reference)
import jax
import jax.numpy as jnp
from jax import lax
from jax.experimental import pallas as pl
from jax.experimental.pallas import tpu as pltpu

N = 100000
C = 1000
B = 4096
TAU = 0.3

_BLK = 256
_GRID = B // _BLK
_Q = 4


def _body(index_sm, idxrp_sm, table, cls_ref, q1_ref, q2_ref, out_ref,
          t1_buf, t2_buf, sem1, sem2):
    i = pl.program_id(0)

    def issue(g, _):
        for k in range(_Q):
            b = g * _Q + k
            gb = i * _BLK + b
            r1 = index_sm[gb]
            r2 = index_sm[idxrp_sm[gb]]
            pltpu.make_async_copy(
                table.at[pl.ds(r1, 1)], t1_buf.at[pl.ds(b, 1)],
                sem1.at[k]).start()
            pltpu.make_async_copy(
                table.at[pl.ds(r2, 1)], t2_buf.at[pl.ds(b, 1)],
                sem2.at[k]).start()
        return 0

    lax.fori_loop(0, _BLK // _Q, issue, 0, unroll=2)

    for k in range(_Q):
        pltpu.make_async_copy(
            table.at[pl.ds(0, _BLK // _Q)],
            t1_buf.at[pl.ds(k * (_BLK // _Q), _BLK // _Q)],
            sem1.at[k]).wait()
        pltpu.make_async_copy(
            table.at[pl.ds(0, _BLK // _Q)],
            t2_buf.at[pl.ds(k * (_BLK // _Q), _BLK // _Q)],
            sem2.at[k]).wait()

    x = cls_ref[...]
    m = jnp.max(x, axis=1, keepdims=True)
    ls = (x - m) - jnp.log(jnp.sum(jnp.exp(x - m), axis=1, keepdims=True))
    y1 = q1_ref[...] * (1.0 / TAU)
    m1 = jnp.max(y1, axis=1, keepdims=True)
    lq1 = (y1 - m1) - jnp.log(jnp.sum(jnp.exp(y1 - m1), axis=1, keepdims=True))
    y2 = q2_ref[...] * (1.0 / TAU)
    m2 = jnp.max(y2, axis=1, keepdims=True)
    lq2 = (y2 - m2) - jnp.log(jnp.sum(jnp.exp(y2 - m2), axis=1, keepdims=True))
    q = lq1 + lq2
    t1 = t1_buf[...]
    t2 = t2_buf[...]
    lt1 = jnp.log(jnp.where(t1 > 0, t1, 1.0))
    lt2 = jnp.log(jnp.where(t2 > 0, t2, 1.0))
    a = jnp.sum(t1 * ls)
    p = jnp.sum(t1 * q)
    r = jnp.sum(t2 * q)
    h1 = jnp.sum(t1 * lt1)
    h2 = jnp.sum(t2 * lt2)
    lane = lax.broadcasted_iota(jnp.int32, (1, 128), 1)
    vec = (jnp.where(lane == 0, a, 0.0) + jnp.where(lane == 1, p, 0.0)
           + jnp.where(lane == 2, r, 0.0) + jnp.where(lane == 3, h1, 0.0)
           + jnp.where(lane == 4, h2, 0.0))

    @pl.when(i == 0)
    def _():
        out_ref[...] = jnp.zeros_like(out_ref)

    out_ref[...] += vec


def _fused(index, idx_rp, table, cls_out1, lpm1, lpm2):
    grid_spec = pltpu.PrefetchScalarGridSpec(
        num_scalar_prefetch=2,
        grid=(_GRID,),
        in_specs=[
            pl.BlockSpec(memory_space=pltpu.MemorySpace.HBM),
            pl.BlockSpec((_BLK, C), lambda i, s1, s2: (i, 0)),
            pl.BlockSpec((_BLK, C), lambda i, s1, s2: (i, 0)),
            pl.BlockSpec((_BLK, C), lambda i, s1, s2: (i, 0)),
        ],
        out_specs=pl.BlockSpec((1, 128), lambda i, s1, s2: (0, 0)),
        scratch_shapes=[
            pltpu.VMEM((_BLK, C), jnp.float32),
            pltpu.VMEM((_BLK, C), jnp.float32),
            pltpu.SemaphoreType.DMA((_Q,)),
            pltpu.SemaphoreType.DMA((_Q,)),
        ],
    )
    return pl.pallas_call(
        _body,
        grid_spec=grid_spec,
        out_shape=jax.ShapeDtypeStruct((1, 128), jnp.float32),
    )(index, idx_rp, table, cls_out1, lpm1, lpm2)


def kernel(predicted_score_cls, cls_out1, cls_out2, logits_prot1,
           logits_prot2, logits_prot_1_mix, logits_prot_2_mix, idx_rp,
           Lambda, index):
    index = index.astype(jnp.int32)
    idx_rp = idx_rp.astype(jnp.int32)
    scal = _fused(index, idx_rp, predicted_score_cls, cls_out1,
                  logits_prot_1_mix, logits_prot_2_mix)
    a, p, r, h1, h2 = scal[0, 0], scal[0, 1], scal[0, 2], scal[0, 3], scal[0, 4]
    bf = jnp.float32(B)
    lam = Lambda.astype(jnp.float32)
    cls_loss_1 = -a / bf
    sim_loss_2 = (2.0 * lam * h1 + 2.0 * (1.0 - lam) * h2
                  - lam * p - (1.0 - lam) * r) / bf
    return (cls_loss_1, sim_loss_2, jnp.float32(1.0))


# speedup vs baseline: 1.0861x; 1.0861x over previous
import jax
import jax.numpy as jnp
from jax import lax
from jax.experimental import pallas as pl
from jax.experimental.pallas import tpu as pltpu

N = 100000
C = 1000
B = 4096
TAU = 0.3

_BLK = 256
_GRID = B // _BLK
_Q = 4


_TBLK = 512


def _transpose_body(tT_ref, out_ref):
    out_ref[...] = jnp.transpose(tT_ref[...], (1, 0))


def _detile(tT):
    return pl.pallas_call(
        _transpose_body,
        grid=(pl.cdiv(N, _TBLK),),
        in_specs=[pl.BlockSpec((C, _TBLK), lambda i: (0, i))],
        out_specs=pl.BlockSpec((_TBLK, C), lambda i: (i, 0)),
        out_shape=jax.ShapeDtypeStruct((N, C), jnp.float32),
    )(tT)


def _body(index_sm, idxrp_sm, table, cls_ref, q1_ref, q2_ref, out_ref,
          t1_buf, t2_buf, sem1, sem2):
    i = pl.program_id(0)

    def issue(g, _):
        for k in range(_Q):
            b = g * _Q + k
            gb = i * _BLK + b
            r1 = index_sm[gb]
            r2 = index_sm[idxrp_sm[gb]]
            pltpu.make_async_copy(
                table.at[pl.ds(r1, 1)], t1_buf.at[pl.ds(b, 1)],
                sem1.at[k]).start()
            pltpu.make_async_copy(
                table.at[pl.ds(r2, 1)], t2_buf.at[pl.ds(b, 1)],
                sem2.at[k]).start()
        return 0

    lax.fori_loop(0, _BLK // _Q, issue, 0, unroll=2)

    for k in range(_Q):
        pltpu.make_async_copy(
            table.at[pl.ds(0, _BLK // _Q)],
            t1_buf.at[pl.ds(k * (_BLK // _Q), _BLK // _Q)],
            sem1.at[k]).wait()
        pltpu.make_async_copy(
            table.at[pl.ds(0, _BLK // _Q)],
            t2_buf.at[pl.ds(k * (_BLK // _Q), _BLK // _Q)],
            sem2.at[k]).wait()

    x = cls_ref[...]
    m = jnp.max(x, axis=1, keepdims=True)
    ls = (x - m) - jnp.log(jnp.sum(jnp.exp(x - m), axis=1, keepdims=True))
    y1 = q1_ref[...] * (1.0 / TAU)
    m1 = jnp.max(y1, axis=1, keepdims=True)
    lq1 = (y1 - m1) - jnp.log(jnp.sum(jnp.exp(y1 - m1), axis=1, keepdims=True))
    y2 = q2_ref[...] * (1.0 / TAU)
    m2 = jnp.max(y2, axis=1, keepdims=True)
    lq2 = (y2 - m2) - jnp.log(jnp.sum(jnp.exp(y2 - m2), axis=1, keepdims=True))
    q = lq1 + lq2
    t1 = t1_buf[...]
    t2 = t2_buf[...]
    lt1 = jnp.log(jnp.where(t1 > 0, t1, 1.0))
    lt2 = jnp.log(jnp.where(t2 > 0, t2, 1.0))
    a = jnp.sum(t1 * ls)
    p = jnp.sum(t1 * q)
    r = jnp.sum(t2 * q)
    h1 = jnp.sum(t1 * lt1)
    h2 = jnp.sum(t2 * lt2)
    lane = lax.broadcasted_iota(jnp.int32, (1, 128), 1)
    vec = (jnp.where(lane == 0, a, 0.0) + jnp.where(lane == 1, p, 0.0)
           + jnp.where(lane == 2, r, 0.0) + jnp.where(lane == 3, h1, 0.0)
           + jnp.where(lane == 4, h2, 0.0))

    @pl.when(i == 0)
    def _():
        out_ref[...] = jnp.zeros_like(out_ref)

    out_ref[...] += vec


def _fused(index, idx_rp, table, cls_out1, lpm1, lpm2):
    grid_spec = pltpu.PrefetchScalarGridSpec(
        num_scalar_prefetch=2,
        grid=(_GRID,),
        in_specs=[
            pl.BlockSpec(memory_space=pltpu.MemorySpace.HBM),
            pl.BlockSpec((_BLK, C), lambda i, s1, s2: (i, 0)),
            pl.BlockSpec((_BLK, C), lambda i, s1, s2: (i, 0)),
            pl.BlockSpec((_BLK, C), lambda i, s1, s2: (i, 0)),
        ],
        out_specs=pl.BlockSpec((1, 128), lambda i, s1, s2: (0, 0)),
        scratch_shapes=[
            pltpu.VMEM((_BLK, C), jnp.float32),
            pltpu.VMEM((_BLK, C), jnp.float32),
            pltpu.SemaphoreType.DMA((_Q,)),
            pltpu.SemaphoreType.DMA((_Q,)),
        ],
    )
    return pl.pallas_call(
        _body,
        grid_spec=grid_spec,
        out_shape=jax.ShapeDtypeStruct((1, 128), jnp.float32),
    )(index, idx_rp, table, cls_out1, lpm1, lpm2)


def kernel(predicted_score_cls, cls_out1, cls_out2, logits_prot1,
           logits_prot2, logits_prot_1_mix, logits_prot_2_mix, idx_rp,
           Lambda, index):
    index = index.astype(jnp.int32)
    idx_rp = idx_rp.astype(jnp.int32)
    table_lin = _detile(jnp.swapaxes(predicted_score_cls, 0, 1))
    scal = _fused(index, idx_rp, table_lin, cls_out1,
                  logits_prot_1_mix, logits_prot_2_mix)
    a, p, r, h1, h2 = scal[0, 0], scal[0, 1], scal[0, 2], scal[0, 3], scal[0, 4]
    bf = jnp.float32(B)
    lam = Lambda.astype(jnp.float32)
    cls_loss_1 = -a / bf
    sim_loss_2 = (2.0 * lam * h1 + 2.0 * (1.0 - lam) * h2
                  - lam * p - (1.0 - lam) * r) / bf
    return (cls_loss_1, sim_loss_2, jnp.float32(1.0))


# speedup vs baseline: 1.2598x; 1.1599x over previous
import jax
import jax.numpy as jnp
from jax import lax
from jax.experimental import pallas as pl
from jax.experimental.pallas import tpu as pltpu

N = 100000
C = 1000
B = 4096
TAU = 0.3

_BLK = 256
_GRID = B // _BLK
_Q = 4


_TBLK = 512


def _transpose_body(tT_ref, out_ref):
    out_ref[...] = jnp.transpose(tT_ref[...], (1, 0))


def _detile(tT):
    return pl.pallas_call(
        _transpose_body,
        grid=(pl.cdiv(N, _TBLK),),
        in_specs=[pl.BlockSpec((C, _TBLK), lambda i: (0, i))],
        out_specs=pl.BlockSpec((_TBLK, C), lambda i: (i, 0)),
        out_shape=jax.ShapeDtypeStruct((N, C), jnp.float32),
    )(tT)


def _body(index_sm, idxrp_sm, table, cls_ref, q1_ref, q2_ref, out_ref,
          t1_buf, t2_buf, sem1, sem2):
    i = pl.program_id(0)

    def issue(step, slot):
        def one(b, _):
            gb = step * _BLK + b
            r1 = index_sm[gb]
            r2 = index_sm[idxrp_sm[gb]]
            pltpu.make_async_copy(
                table.at[pl.ds(r1, 1)], t1_buf.at[slot, pl.ds(b, 1)],
                sem1.at[slot]).start()
            pltpu.make_async_copy(
                table.at[pl.ds(r2, 1)], t2_buf.at[slot, pl.ds(b, 1)],
                sem2.at[slot]).start()
            return 0

        lax.fori_loop(0, _BLK, one, 0, unroll=8)

    slot = lax.rem(i, 2)

    @pl.when(i == 0)
    def _():
        issue(0, 0)

    pltpu.make_async_copy(
        table.at[pl.ds(0, _BLK)], t1_buf.at[slot], sem1.at[slot]).wait()
    pltpu.make_async_copy(
        table.at[pl.ds(0, _BLK)], t2_buf.at[slot], sem2.at[slot]).wait()

    @pl.when(i + 1 < _GRID)
    def _():
        issue(i + 1, 1 - slot)

    x = cls_ref[...]
    m = jnp.max(x, axis=0, keepdims=True)
    ls = (x - m) - jnp.log(jnp.sum(jnp.exp(x - m), axis=0, keepdims=True))
    y1 = q1_ref[...] * (1.0 / TAU)
    m1 = jnp.max(y1, axis=0, keepdims=True)
    lq1 = (y1 - m1) - jnp.log(jnp.sum(jnp.exp(y1 - m1), axis=0, keepdims=True))
    y2 = q2_ref[...] * (1.0 / TAU)
    m2 = jnp.max(y2, axis=0, keepdims=True)
    lq2 = (y2 - m2) - jnp.log(jnp.sum(jnp.exp(y2 - m2), axis=0, keepdims=True))
    q = lq1 + lq2
    t1 = jnp.transpose(t1_buf[slot], (1, 0))
    t2 = jnp.transpose(t2_buf[slot], (1, 0))
    lt1 = jnp.log(jnp.where(t1 > 0, t1, 1.0))
    lt2 = jnp.log(jnp.where(t2 > 0, t2, 1.0))
    a = jnp.sum(t1 * ls)
    p = jnp.sum(t1 * q)
    r = jnp.sum(t2 * q)
    h1 = jnp.sum(t1 * lt1)
    h2 = jnp.sum(t2 * lt2)
    lane = lax.broadcasted_iota(jnp.int32, (1, 128), 1)
    vec = (jnp.where(lane == 0, a, 0.0) + jnp.where(lane == 1, p, 0.0)
           + jnp.where(lane == 2, r, 0.0) + jnp.where(lane == 3, h1, 0.0)
           + jnp.where(lane == 4, h2, 0.0))

    @pl.when(i == 0)
    def _():
        out_ref[...] = jnp.zeros_like(out_ref)

    out_ref[...] += vec


def _fused(index, idx_rp, table, cls_out1, lpm1, lpm2):
    grid_spec = pltpu.PrefetchScalarGridSpec(
        num_scalar_prefetch=2,
        grid=(_GRID,),
        in_specs=[
            pl.BlockSpec(memory_space=pltpu.MemorySpace.HBM),
            pl.BlockSpec((C, _BLK), lambda i, s1, s2: (0, i)),
            pl.BlockSpec((C, _BLK), lambda i, s1, s2: (0, i)),
            pl.BlockSpec((C, _BLK), lambda i, s1, s2: (0, i)),
        ],
        out_specs=pl.BlockSpec((1, 128), lambda i, s1, s2: (0, 0)),
        scratch_shapes=[
            pltpu.VMEM((2, _BLK, C), jnp.float32),
            pltpu.VMEM((2, _BLK, C), jnp.float32),
            pltpu.SemaphoreType.DMA((2,)),
            pltpu.SemaphoreType.DMA((2,)),
        ],
    )
    return pl.pallas_call(
        _body,
        grid_spec=grid_spec,
        out_shape=jax.ShapeDtypeStruct((1, 128), jnp.float32),
    )(index, idx_rp, table, cls_out1, lpm1, lpm2)


def kernel(predicted_score_cls, cls_out1, cls_out2, logits_prot1,
           logits_prot2, logits_prot_1_mix, logits_prot_2_mix, idx_rp,
           Lambda, index):
    index = index.astype(jnp.int32)
    idx_rp = idx_rp.astype(jnp.int32)
    table_lin = _detile(jnp.swapaxes(predicted_score_cls, 0, 1))
    scal = _fused(index, idx_rp, table_lin,
                  jnp.swapaxes(cls_out1, 0, 1),
                  jnp.swapaxes(logits_prot_1_mix, 0, 1),
                  jnp.swapaxes(logits_prot_2_mix, 0, 1))
    a, p, r, h1, h2 = scal[0, 0], scal[0, 1], scal[0, 2], scal[0, 3], scal[0, 4]
    bf = jnp.float32(B)
    lam = Lambda.astype(jnp.float32)
    cls_loss_1 = -a / bf
    sim_loss_2 = (2.0 * lam * h1 + 2.0 * (1.0 - lam) * h2
                  - lam * p - (1.0 - lam) * r) / bf
    return (cls_loss_1, sim_loss_2, jnp.float32(1.0))


# speedup vs baseline: 1.4002x; 1.1114x over previous
import jax
import jax.numpy as jnp
from jax import lax
from jax.experimental import pallas as pl
from jax.experimental.pallas import tpu as pltpu

N = 100000
C = 1000
B = 4096
TAU = 0.3

_BLK = 256
_GRID = B // _BLK
_Q = 4


_TBLK = 512
_PW = 384
_CP = _PW + (C - 2 * _PW)


def _transpose_body(tT_ref, out_ref):
    t = jnp.transpose(tT_ref[...], (1, 0))
    lo = lax.bitcast_convert_type(
        t[:, 0:_PW].astype(jnp.bfloat16), jnp.uint16).astype(jnp.uint32)
    hi = lax.bitcast_convert_type(
        t[:, _PW:2 * _PW].astype(jnp.bfloat16), jnp.uint16).astype(jnp.uint32)
    packed = lax.bitcast_convert_type(lo | (hi << 16), jnp.float32)
    out_ref[...] = jnp.concatenate([packed, t[:, 2 * _PW:C]], axis=1)


def _detile(tT):
    return pl.pallas_call(
        _transpose_body,
        grid=(pl.cdiv(N, _TBLK),),
        in_specs=[pl.BlockSpec((C, _TBLK), lambda i: (0, i))],
        out_specs=pl.BlockSpec((_TBLK, _CP), lambda i: (i, 0)),
        out_shape=jax.ShapeDtypeStruct((N, _CP), jnp.float32),
    )(tT)


def _unpack_rows(p):
    pu = lax.bitcast_convert_type(p[:, 0:_PW], jnp.uint32)
    lo = lax.bitcast_convert_type(
        (pu & jnp.uint32(0xFFFF)).astype(jnp.uint16), jnp.bfloat16)
    hi = lax.bitcast_convert_type(
        (pu >> jnp.uint32(16)).astype(jnp.uint16), jnp.bfloat16)
    return jnp.concatenate(
        [lo.astype(jnp.float32), hi.astype(jnp.float32), p[:, _PW:_CP]],
        axis=1)


def _body(index_sm, idxrp_sm, table, cls_ref, q1_ref, q2_ref, out_ref,
          t1_buf, t2_buf, sem1, sem2):
    i = pl.program_id(0)

    def issue(step, slot):
        def one(b, _):
            gb = step * _BLK + b
            r1 = index_sm[gb]
            r2 = index_sm[idxrp_sm[gb]]
            pltpu.make_async_copy(
                table.at[pl.ds(r1, 1)], t1_buf.at[slot, pl.ds(b, 1)],
                sem1.at[slot]).start()
            pltpu.make_async_copy(
                table.at[pl.ds(r2, 1)], t2_buf.at[slot, pl.ds(b, 1)],
                sem2.at[slot]).start()
            return 0

        lax.fori_loop(0, _BLK, one, 0, unroll=8)

    slot = lax.rem(i, 2)

    @pl.when(i == 0)
    def _():
        issue(0, 0)

    pltpu.make_async_copy(
        table.at[pl.ds(0, _BLK)], t1_buf.at[slot], sem1.at[slot]).wait()
    pltpu.make_async_copy(
        table.at[pl.ds(0, _BLK)], t2_buf.at[slot], sem2.at[slot]).wait()

    @pl.when(i + 1 < _GRID)
    def _():
        issue(i + 1, 1 - slot)

    x = cls_ref[...]
    m = jnp.max(x, axis=0, keepdims=True)
    ls = (x - m) - jnp.log(jnp.sum(jnp.exp(x - m), axis=0, keepdims=True))
    y1 = q1_ref[...] * (1.0 / TAU)
    m1 = jnp.max(y1, axis=0, keepdims=True)
    lq1 = (y1 - m1) - jnp.log(jnp.sum(jnp.exp(y1 - m1), axis=0, keepdims=True))
    y2 = q2_ref[...] * (1.0 / TAU)
    m2 = jnp.max(y2, axis=0, keepdims=True)
    lq2 = (y2 - m2) - jnp.log(jnp.sum(jnp.exp(y2 - m2), axis=0, keepdims=True))
    q = lq1 + lq2
    t1 = jnp.transpose(_unpack_rows(t1_buf[slot]), (1, 0))
    t2 = jnp.transpose(_unpack_rows(t2_buf[slot]), (1, 0))
    lt1 = jnp.log(jnp.where(t1 > 0, t1, 1.0))
    lt2 = jnp.log(jnp.where(t2 > 0, t2, 1.0))
    a = jnp.sum(t1 * ls)
    p = jnp.sum(t1 * q)
    r = jnp.sum(t2 * q)
    h1 = jnp.sum(t1 * lt1)
    h2 = jnp.sum(t2 * lt2)
    lane = lax.broadcasted_iota(jnp.int32, (1, 128), 1)
    vec = (jnp.where(lane == 0, a, 0.0) + jnp.where(lane == 1, p, 0.0)
           + jnp.where(lane == 2, r, 0.0) + jnp.where(lane == 3, h1, 0.0)
           + jnp.where(lane == 4, h2, 0.0))

    @pl.when(i == 0)
    def _():
        out_ref[...] = jnp.zeros_like(out_ref)

    out_ref[...] += vec


def _fused(index, idx_rp, table, cls_out1, lpm1, lpm2):
    grid_spec = pltpu.PrefetchScalarGridSpec(
        num_scalar_prefetch=2,
        grid=(_GRID,),
        in_specs=[
            pl.BlockSpec(memory_space=pltpu.MemorySpace.HBM),
            pl.BlockSpec((C, _BLK), lambda i, s1, s2: (0, i)),
            pl.BlockSpec((C, _BLK), lambda i, s1, s2: (0, i)),
            pl.BlockSpec((C, _BLK), lambda i, s1, s2: (0, i)),
        ],
        out_specs=pl.BlockSpec((1, 128), lambda i, s1, s2: (0, 0)),
        scratch_shapes=[
            pltpu.VMEM((2, _BLK, _CP), jnp.float32),
            pltpu.VMEM((2, _BLK, _CP), jnp.float32),
            pltpu.SemaphoreType.DMA((2,)),
            pltpu.SemaphoreType.DMA((2,)),
        ],
    )
    return pl.pallas_call(
        _body,
        grid_spec=grid_spec,
        out_shape=jax.ShapeDtypeStruct((1, 128), jnp.float32),
    )(index, idx_rp, table, cls_out1, lpm1, lpm2)


def kernel(predicted_score_cls, cls_out1, cls_out2, logits_prot1,
           logits_prot2, logits_prot_1_mix, logits_prot_2_mix, idx_rp,
           Lambda, index):
    index = index.astype(jnp.int32)
    idx_rp = idx_rp.astype(jnp.int32)
    table_lin = _detile(jnp.swapaxes(predicted_score_cls, 0, 1))
    scal = _fused(index, idx_rp, table_lin,
                  jnp.swapaxes(cls_out1, 0, 1),
                  jnp.swapaxes(logits_prot_1_mix, 0, 1),
                  jnp.swapaxes(logits_prot_2_mix, 0, 1))
    a, p, r, h1, h2 = scal[0, 0], scal[0, 1], scal[0, 2], scal[0, 3], scal[0, 4]
    bf = jnp.float32(B)
    lam = Lambda.astype(jnp.float32)
    cls_loss_1 = -a / bf
    sim_loss_2 = (2.0 * lam * h1 + 2.0 * (1.0 - lam) * h2
                  - lam * p - (1.0 - lam) * r) / bf
    return (cls_loss_1, sim_loss_2, jnp.float32(1.0))


# speedup vs baseline: 1.6404x; 1.1716x over previous
import jax
import jax.numpy as jnp
from jax import lax
from jax.experimental import pallas as pl
from jax.experimental.pallas import tpu as pltpu

N = 100000
C = 1000
B = 4096
TAU = 0.3

_BLK = 256
_GRID = B // _BLK
_Q = 4


_TBLK = 1024
_PW = 384
_CP = _PW + (C - 2 * _PW)


def _transpose_body(tT_ref, out_ref):
    t = jnp.transpose(tT_ref[...], (1, 0))
    lo = lax.bitcast_convert_type(
        t[:, 0:_PW].astype(jnp.bfloat16), jnp.uint16).astype(jnp.uint32)
    hi = lax.bitcast_convert_type(
        t[:, _PW:2 * _PW].astype(jnp.bfloat16), jnp.uint16).astype(jnp.uint32)
    packed = lax.bitcast_convert_type(lo | (hi << 16), jnp.float32)
    out_ref[...] = jnp.concatenate([packed, t[:, 2 * _PW:C]], axis=1)


def _detile(tT):
    return pl.pallas_call(
        _transpose_body,
        grid=(pl.cdiv(N, _TBLK),),
        in_specs=[pl.BlockSpec((C, _TBLK), lambda i: (0, i))],
        out_specs=pl.BlockSpec((_TBLK, _CP), lambda i: (i, 0)),
        out_shape=jax.ShapeDtypeStruct((N, _CP), jnp.float32),
    )(tT)


def _unpack_rows(p):
    pu = lax.bitcast_convert_type(p[:, 0:_PW], jnp.uint32)
    lo = lax.bitcast_convert_type(
        (pu & jnp.uint32(0xFFFF)).astype(jnp.uint16), jnp.bfloat16)
    hi = lax.bitcast_convert_type(
        (pu >> jnp.uint32(16)).astype(jnp.uint16), jnp.bfloat16)
    return jnp.concatenate(
        [lo.astype(jnp.float32), hi.astype(jnp.float32), p[:, _PW:_CP]],
        axis=1)


def _body(index_sm, idxrp_sm, table, cls_ref, q1_ref, q2_ref, out_ref,
          t1_buf, t2_buf, sem1, sem2):
    i = pl.program_id(0)

    def issue(step, slot):
        def one(b, _):
            gb = step * _BLK + b
            r1 = index_sm[gb]
            r2 = index_sm[idxrp_sm[gb]]
            pltpu.make_async_copy(
                table.at[pl.ds(r1, 1)], t1_buf.at[slot, pl.ds(b, 1)],
                sem1.at[slot]).start()
            pltpu.make_async_copy(
                table.at[pl.ds(r2, 1)], t2_buf.at[slot, pl.ds(b, 1)],
                sem2.at[slot]).start()
            return 0

        lax.fori_loop(0, _BLK, one, 0, unroll=8)

    slot = lax.rem(i, 2)

    @pl.when(i == 0)
    def _():
        issue(0, 0)

    pltpu.make_async_copy(
        table.at[pl.ds(0, _BLK)], t1_buf.at[slot], sem1.at[slot]).wait()
    pltpu.make_async_copy(
        table.at[pl.ds(0, _BLK)], t2_buf.at[slot], sem2.at[slot]).wait()

    @pl.when(i + 1 < _GRID)
    def _():
        issue(i + 1, 1 - slot)

    x = cls_ref[...]
    m = jnp.max(x, axis=0, keepdims=True)
    ls = (x - m) - jnp.log(jnp.sum(jnp.exp(x - m), axis=0, keepdims=True))
    y1 = q1_ref[...] * (1.0 / TAU)
    m1 = jnp.max(y1, axis=0, keepdims=True)
    lq1 = (y1 - m1) - jnp.log(jnp.sum(jnp.exp(y1 - m1), axis=0, keepdims=True))
    y2 = q2_ref[...] * (1.0 / TAU)
    m2 = jnp.max(y2, axis=0, keepdims=True)
    lq2 = (y2 - m2) - jnp.log(jnp.sum(jnp.exp(y2 - m2), axis=0, keepdims=True))
    q = lq1 + lq2
    t1 = jnp.transpose(_unpack_rows(t1_buf[slot]), (1, 0))
    t2 = jnp.transpose(_unpack_rows(t2_buf[slot]), (1, 0))
    lt1 = jnp.log(jnp.where(t1 > 0, t1, 1.0))
    lt2 = jnp.log(jnp.where(t2 > 0, t2, 1.0))
    a = jnp.sum(t1 * ls)
    p = jnp.sum(t1 * q)
    r = jnp.sum(t2 * q)
    h1 = jnp.sum(t1 * lt1)
    h2 = jnp.sum(t2 * lt2)
    lane = lax.broadcasted_iota(jnp.int32, (1, 128), 1)
    vec = (jnp.where(lane == 0, a, 0.0) + jnp.where(lane == 1, p, 0.0)
           + jnp.where(lane == 2, r, 0.0) + jnp.where(lane == 3, h1, 0.0)
           + jnp.where(lane == 4, h2, 0.0))

    @pl.when(i == 0)
    def _():
        out_ref[...] = jnp.zeros_like(out_ref)

    out_ref[...] += vec


def _fused(index, idx_rp, table, cls_out1, lpm1, lpm2):
    grid_spec = pltpu.PrefetchScalarGridSpec(
        num_scalar_prefetch=2,
        grid=(_GRID,),
        in_specs=[
            pl.BlockSpec(memory_space=pltpu.MemorySpace.HBM),
            pl.BlockSpec((C, _BLK), lambda i, s1, s2: (0, i)),
            pl.BlockSpec((C, _BLK), lambda i, s1, s2: (0, i)),
            pl.BlockSpec((C, _BLK), lambda i, s1, s2: (0, i)),
        ],
        out_specs=pl.BlockSpec((1, 128), lambda i, s1, s2: (0, 0)),
        scratch_shapes=[
            pltpu.VMEM((2, _BLK, _CP), jnp.float32),
            pltpu.VMEM((2, _BLK, _CP), jnp.float32),
            pltpu.SemaphoreType.DMA((2,)),
            pltpu.SemaphoreType.DMA((2,)),
        ],
    )
    return pl.pallas_call(
        _body,
        grid_spec=grid_spec,
        out_shape=jax.ShapeDtypeStruct((1, 128), jnp.float32),
    )(index, idx_rp, table, cls_out1, lpm1, lpm2)


def kernel(predicted_score_cls, cls_out1, cls_out2, logits_prot1,
           logits_prot2, logits_prot_1_mix, logits_prot_2_mix, idx_rp,
           Lambda, index):
    index = index.astype(jnp.int32)
    idx_rp = idx_rp.astype(jnp.int32)
    table_lin = _detile(jnp.swapaxes(predicted_score_cls, 0, 1))
    scal = _fused(index, idx_rp, table_lin,
                  jnp.swapaxes(cls_out1, 0, 1),
                  jnp.swapaxes(logits_prot_1_mix, 0, 1),
                  jnp.swapaxes(logits_prot_2_mix, 0, 1))
    a, p, r, h1, h2 = scal[0, 0], scal[0, 1], scal[0, 2], scal[0, 3], scal[0, 4]
    bf = jnp.float32(B)
    lam = Lambda.astype(jnp.float32)
    cls_loss_1 = -a / bf
    sim_loss_2 = (2.0 * lam * h1 + 2.0 * (1.0 - lam) * h2
                  - lam * p - (1.0 - lam) * r) / bf
    return (cls_loss_1, sim_loss_2, jnp.float32(1.0))


# speedup vs baseline: 1.6954x; 1.0335x over previous
import jax
import jax.numpy as jnp
from jax import lax
from jax.experimental import pallas as pl
from jax.experimental.pallas import tpu as pltpu

N = 100000
C = 1000
B = 4096
TAU = 0.3

_BLK = 256
_GRID = B // _BLK
_Q = 4


_TBLK = 2048
_PW = 384
_CP = _PW + (C - 2 * _PW)


def _transpose_body(tT_ref, out_ref):
    t = jnp.transpose(tT_ref[...], (1, 0))
    lo = lax.bitcast_convert_type(
        t[:, 0:_PW].astype(jnp.bfloat16), jnp.uint16).astype(jnp.uint32)
    hi = lax.bitcast_convert_type(
        t[:, _PW:2 * _PW].astype(jnp.bfloat16), jnp.uint16).astype(jnp.uint32)
    packed = lax.bitcast_convert_type(lo | (hi << 16), jnp.float32)
    out_ref[...] = jnp.concatenate([packed, t[:, 2 * _PW:C]], axis=1)


def _detile(tT):
    return pl.pallas_call(
        _transpose_body,
        grid=(pl.cdiv(N, _TBLK),),
        in_specs=[pl.BlockSpec((C, _TBLK), lambda i: (0, i))],
        out_specs=pl.BlockSpec((_TBLK, _CP), lambda i: (i, 0)),
        out_shape=jax.ShapeDtypeStruct((N, _CP), jnp.float32),
    )(tT)


def _unpack_rows(p):
    pu = lax.bitcast_convert_type(p[:, 0:_PW], jnp.uint32)
    lo = lax.bitcast_convert_type(
        (pu & jnp.uint32(0xFFFF)).astype(jnp.uint16), jnp.bfloat16)
    hi = lax.bitcast_convert_type(
        (pu >> jnp.uint32(16)).astype(jnp.uint16), jnp.bfloat16)
    return jnp.concatenate(
        [lo.astype(jnp.float32), hi.astype(jnp.float32), p[:, _PW:_CP]],
        axis=1)


def _body(index_sm, idxrp_sm, table, cls_ref, q1_ref, q2_ref, out_ref,
          t1_buf, t2_buf, sem1, sem2):
    i = pl.program_id(0)

    def issue(step, slot):
        def one(b, _):
            gb = step * _BLK + b
            r1 = index_sm[gb]
            r2 = index_sm[idxrp_sm[gb]]
            pltpu.make_async_copy(
                table.at[pl.ds(r1, 1)], t1_buf.at[slot, pl.ds(b, 1)],
                sem1.at[slot]).start()
            pltpu.make_async_copy(
                table.at[pl.ds(r2, 1)], t2_buf.at[slot, pl.ds(b, 1)],
                sem2.at[slot]).start()
            return 0

        lax.fori_loop(0, _BLK, one, 0, unroll=8)

    slot = lax.rem(i, 2)

    @pl.when(i == 0)
    def _():
        issue(0, 0)

    pltpu.make_async_copy(
        table.at[pl.ds(0, _BLK)], t1_buf.at[slot], sem1.at[slot]).wait()
    pltpu.make_async_copy(
        table.at[pl.ds(0, _BLK)], t2_buf.at[slot], sem2.at[slot]).wait()

    @pl.when(i + 1 < _GRID)
    def _():
        issue(i + 1, 1 - slot)

    x = cls_ref[...]
    m = jnp.max(x, axis=0, keepdims=True)
    ls = (x - m) - jnp.log(jnp.sum(jnp.exp(x - m), axis=0, keepdims=True))
    y1 = q1_ref[...] * (1.0 / TAU)
    m1 = jnp.max(y1, axis=0, keepdims=True)
    lq1 = (y1 - m1) - jnp.log(jnp.sum(jnp.exp(y1 - m1), axis=0, keepdims=True))
    y2 = q2_ref[...] * (1.0 / TAU)
    m2 = jnp.max(y2, axis=0, keepdims=True)
    lq2 = (y2 - m2) - jnp.log(jnp.sum(jnp.exp(y2 - m2), axis=0, keepdims=True))
    q = lq1 + lq2
    t1 = jnp.transpose(_unpack_rows(t1_buf[slot]), (1, 0))
    t2 = jnp.transpose(_unpack_rows(t2_buf[slot]), (1, 0))
    lt1 = jnp.log(jnp.where(t1 > 0, t1, 1.0))
    lt2 = jnp.log(jnp.where(t2 > 0, t2, 1.0))
    a = jnp.sum(t1 * ls)
    p = jnp.sum(t1 * q)
    r = jnp.sum(t2 * q)
    h1 = jnp.sum(t1 * lt1)
    h2 = jnp.sum(t2 * lt2)
    lane = lax.broadcasted_iota(jnp.int32, (1, 128), 1)
    vec = (jnp.where(lane == 0, a, 0.0) + jnp.where(lane == 1, p, 0.0)
           + jnp.where(lane == 2, r, 0.0) + jnp.where(lane == 3, h1, 0.0)
           + jnp.where(lane == 4, h2, 0.0))

    @pl.when(i == 0)
    def _():
        out_ref[...] = jnp.zeros_like(out_ref)

    out_ref[...] += vec


def _fused(index, idx_rp, table, cls_out1, lpm1, lpm2):
    grid_spec = pltpu.PrefetchScalarGridSpec(
        num_scalar_prefetch=2,
        grid=(_GRID,),
        in_specs=[
            pl.BlockSpec(memory_space=pltpu.MemorySpace.HBM),
            pl.BlockSpec((C, _BLK), lambda i, s1, s2: (0, i)),
            pl.BlockSpec((C, _BLK), lambda i, s1, s2: (0, i)),
            pl.BlockSpec((C, _BLK), lambda i, s1, s2: (0, i)),
        ],
        out_specs=pl.BlockSpec((1, 128), lambda i, s1, s2: (0, 0)),
        scratch_shapes=[
            pltpu.VMEM((2, _BLK, _CP), jnp.float32),
            pltpu.VMEM((2, _BLK, _CP), jnp.float32),
            pltpu.SemaphoreType.DMA((2,)),
            pltpu.SemaphoreType.DMA((2,)),
        ],
    )
    return pl.pallas_call(
        _body,
        grid_spec=grid_spec,
        out_shape=jax.ShapeDtypeStruct((1, 128), jnp.float32),
    )(index, idx_rp, table, cls_out1, lpm1, lpm2)


def kernel(predicted_score_cls, cls_out1, cls_out2, logits_prot1,
           logits_prot2, logits_prot_1_mix, logits_prot_2_mix, idx_rp,
           Lambda, index):
    index = index.astype(jnp.int32)
    idx_rp = idx_rp.astype(jnp.int32)
    table_lin = _detile(jnp.swapaxes(predicted_score_cls, 0, 1))
    scal = _fused(index, idx_rp, table_lin,
                  jnp.swapaxes(cls_out1, 0, 1),
                  jnp.swapaxes(logits_prot_1_mix, 0, 1),
                  jnp.swapaxes(logits_prot_2_mix, 0, 1))
    a, p, r, h1, h2 = scal[0, 0], scal[0, 1], scal[0, 2], scal[0, 3], scal[0, 4]
    bf = jnp.float32(B)
    lam = Lambda.astype(jnp.float32)
    cls_loss_1 = -a / bf
    sim_loss_2 = (2.0 * lam * h1 + 2.0 * (1.0 - lam) * h2
                  - lam * p - (1.0 - lam) * r) / bf
    return (cls_loss_1, sim_loss_2, jnp.float32(1.0))


# speedup vs baseline: 1.7334x; 1.0224x over previous
import jax
import jax.numpy as jnp
from jax import lax
from jax.experimental import pallas as pl
from jax.experimental.pallas import tpu as pltpu

N = 100000
C = 1000
B = 4096
TAU = 0.3

_BLK = 512
_GRID = B // _BLK
_Q = 4


_TBLK = 4096
_PW = 384
_CP = _PW + (C - 2 * _PW)


def _transpose_body(tT_ref, out_ref):
    t = jnp.transpose(tT_ref[...], (1, 0))
    lo = lax.bitcast_convert_type(
        t[:, 0:_PW].astype(jnp.bfloat16), jnp.uint16).astype(jnp.uint32)
    hi = lax.bitcast_convert_type(
        t[:, _PW:2 * _PW].astype(jnp.bfloat16), jnp.uint16).astype(jnp.uint32)
    packed = lax.bitcast_convert_type(lo | (hi << 16), jnp.float32)
    out_ref[...] = jnp.concatenate([packed, t[:, 2 * _PW:C]], axis=1)


def _detile(tT):
    return pl.pallas_call(
        _transpose_body,
        grid=(pl.cdiv(N, _TBLK),),
        in_specs=[pl.BlockSpec((C, _TBLK), lambda i: (0, i))],
        out_specs=pl.BlockSpec((_TBLK, _CP), lambda i: (i, 0)),
        out_shape=jax.ShapeDtypeStruct((N, _CP), jnp.float32),
    )(tT)


def _unpack_rows(p):
    pu = lax.bitcast_convert_type(p[:, 0:_PW], jnp.uint32)
    lo = lax.bitcast_convert_type(
        (pu & jnp.uint32(0xFFFF)).astype(jnp.uint16), jnp.bfloat16)
    hi = lax.bitcast_convert_type(
        (pu >> jnp.uint32(16)).astype(jnp.uint16), jnp.bfloat16)
    return jnp.concatenate(
        [lo.astype(jnp.float32), hi.astype(jnp.float32), p[:, _PW:_CP]],
        axis=1)


def _body(index_sm, idxrp_sm, table, cls_ref, q1_ref, q2_ref, out_ref,
          t1_buf, t2_buf, sem1, sem2):
    i = pl.program_id(0)

    def issue(step, slot):
        def one(b, _):
            gb = step * _BLK + b
            r1 = index_sm[gb]
            r2 = index_sm[idxrp_sm[gb]]
            pltpu.make_async_copy(
                table.at[pl.ds(r1, 1)], t1_buf.at[slot, pl.ds(b, 1)],
                sem1.at[slot]).start()
            pltpu.make_async_copy(
                table.at[pl.ds(r2, 1)], t2_buf.at[slot, pl.ds(b, 1)],
                sem2.at[slot]).start()
            return 0

        lax.fori_loop(0, _BLK, one, 0, unroll=8)

    slot = lax.rem(i, 2)

    @pl.when(i == 0)
    def _():
        issue(0, 0)

    pltpu.make_async_copy(
        table.at[pl.ds(0, _BLK)], t1_buf.at[slot], sem1.at[slot]).wait()
    pltpu.make_async_copy(
        table.at[pl.ds(0, _BLK)], t2_buf.at[slot], sem2.at[slot]).wait()

    @pl.when(i + 1 < _GRID)
    def _():
        issue(i + 1, 1 - slot)

    x = cls_ref[...]
    m = jnp.max(x, axis=0, keepdims=True)
    ls = (x - m) - jnp.log(jnp.sum(jnp.exp(x - m), axis=0, keepdims=True))
    y1 = q1_ref[...] * (1.0 / TAU)
    m1 = jnp.max(y1, axis=0, keepdims=True)
    lq1 = (y1 - m1) - jnp.log(jnp.sum(jnp.exp(y1 - m1), axis=0, keepdims=True))
    y2 = q2_ref[...] * (1.0 / TAU)
    m2 = jnp.max(y2, axis=0, keepdims=True)
    lq2 = (y2 - m2) - jnp.log(jnp.sum(jnp.exp(y2 - m2), axis=0, keepdims=True))
    q = lq1 + lq2
    t1 = jnp.transpose(_unpack_rows(t1_buf[slot]), (1, 0))
    t2 = jnp.transpose(_unpack_rows(t2_buf[slot]), (1, 0))
    lt1 = jnp.log(jnp.where(t1 > 0, t1, 1.0))
    lt2 = jnp.log(jnp.where(t2 > 0, t2, 1.0))
    a = jnp.sum(t1 * ls)
    p = jnp.sum(t1 * q)
    r = jnp.sum(t2 * q)
    h1 = jnp.sum(t1 * lt1)
    h2 = jnp.sum(t2 * lt2)
    lane = lax.broadcasted_iota(jnp.int32, (1, 128), 1)
    vec = (jnp.where(lane == 0, a, 0.0) + jnp.where(lane == 1, p, 0.0)
           + jnp.where(lane == 2, r, 0.0) + jnp.where(lane == 3, h1, 0.0)
           + jnp.where(lane == 4, h2, 0.0))

    @pl.when(i == 0)
    def _():
        out_ref[...] = jnp.zeros_like(out_ref)

    out_ref[...] += vec


def _fused(index, idx_rp, table, cls_out1, lpm1, lpm2):
    grid_spec = pltpu.PrefetchScalarGridSpec(
        num_scalar_prefetch=2,
        grid=(_GRID,),
        in_specs=[
            pl.BlockSpec(memory_space=pltpu.MemorySpace.HBM),
            pl.BlockSpec((C, _BLK), lambda i, s1, s2: (0, i)),
            pl.BlockSpec((C, _BLK), lambda i, s1, s2: (0, i)),
            pl.BlockSpec((C, _BLK), lambda i, s1, s2: (0, i)),
        ],
        out_specs=pl.BlockSpec((1, 128), lambda i, s1, s2: (0, 0)),
        scratch_shapes=[
            pltpu.VMEM((2, _BLK, _CP), jnp.float32),
            pltpu.VMEM((2, _BLK, _CP), jnp.float32),
            pltpu.SemaphoreType.DMA((2,)),
            pltpu.SemaphoreType.DMA((2,)),
        ],
    )
    return pl.pallas_call(
        _body,
        grid_spec=grid_spec,
        out_shape=jax.ShapeDtypeStruct((1, 128), jnp.float32),
    )(index, idx_rp, table, cls_out1, lpm1, lpm2)


def kernel(predicted_score_cls, cls_out1, cls_out2, logits_prot1,
           logits_prot2, logits_prot_1_mix, logits_prot_2_mix, idx_rp,
           Lambda, index):
    index = index.astype(jnp.int32)
    idx_rp = idx_rp.astype(jnp.int32)
    table_lin = _detile(jnp.swapaxes(predicted_score_cls, 0, 1))
    scal = _fused(index, idx_rp, table_lin,
                  jnp.swapaxes(cls_out1, 0, 1),
                  jnp.swapaxes(logits_prot_1_mix, 0, 1),
                  jnp.swapaxes(logits_prot_2_mix, 0, 1))
    a, p, r, h1, h2 = scal[0, 0], scal[0, 1], scal[0, 2], scal[0, 3], scal[0, 4]
    bf = jnp.float32(B)
    lam = Lambda.astype(jnp.float32)
    cls_loss_1 = -a / bf
    sim_loss_2 = (2.0 * lam * h1 + 2.0 * (1.0 - lam) * h2
                  - lam * p - (1.0 - lam) * r) / bf
    return (cls_loss_1, sim_loss_2, jnp.float32(1.0))
